# baseline (speedup 1.0000x reference)
import jax
import jax.numpy as jnp
from jax.experimental import pallas as pl
from jax.experimental.pallas import tpu as pltpu

E, N, D = 8, 4096, 1024
TN = 1024


def _moe_body(x_ref, wr_ref, br_ref, we_ref, be_ref, out_ref, gn_ref):
    e = pl.program_id(1)
    xb = x_ref[0]

    @pl.when(e == 0)
    def _init():
        logits = (
            jnp.dot(xb, wr_ref[...], preferred_element_type=jnp.float32)
            + br_ref[...]
        )
        m = jnp.max(logits, axis=-1, keepdims=True)
        ex = jnp.exp(logits - m)
        gates = ex / jnp.sum(ex, axis=-1, keepdims=True)
        gn = gates / jnp.sum(gates, axis=-1, keepdims=True)
        gn_ref[...] = gn
        out_ref[...] = jnp.dot(gn, be_ref[...], preferred_element_type=jnp.float32)

    onehot = (jax.lax.broadcasted_iota(jnp.int32, (1, E), 1) == e).astype(
        jnp.float32
    )
    gcol = jnp.sum(gn_ref[...] * onehot, axis=-1, keepdims=True)

    partial = jnp.dot(xb, we_ref[e], preferred_element_type=jnp.float32)
    out_ref[...] += gcol * partial


@jax.jit
def _moe(x, Wr, br, We, be):
    num_tiles = N // TN
    return pl.pallas_call(
        _moe_body,
        grid=(num_tiles, E),
        in_specs=[
            pl.BlockSpec((1, TN, D), lambda nt, e: (e, nt, 0)),
            pl.BlockSpec((D, E), lambda nt, e: (0, 0)),
            pl.BlockSpec((1, E), lambda nt, e: (0, 0)),
            pl.BlockSpec((E, D, D), lambda nt, e: (0, 0, 0)),
            pl.BlockSpec((E, D), lambda nt, e: (0, 0)),
        ],
        out_specs=pl.BlockSpec((TN, D), lambda nt, e: (nt, 0)),
        out_shape=jax.ShapeDtypeStruct((N, D), jnp.float32),
        scratch_shapes=[pltpu.VMEM((TN, E), jnp.float32)],
        compiler_params=pltpu.CompilerParams(
            dimension_semantics=("arbitrary", "arbitrary"),
        ),
    )(x, Wr, br, We, be)


def kernel(x, Wr, br, We, be):
    return _moe(x, Wr, br.reshape(1, E), We, be)
